# Initial kernel scaffold; baseline (speedup 1.0000x reference)
import functools

import jax
import jax.numpy as jnp
from jax import lax
from jax.experimental import pallas as pl
from jax.experimental.pallas import tpu as pltpu
from jax.experimental.pallas import tpu_sc as plsc

NC, NS, L = 2, 16, 16
NW = NC * NS


def _bracket_sc(u1w, u2w, idx_i, idx_j, idx_k, coef, D, BC, NNZ):
    T = BC // L
    mesh = plsc.VectorSubcoreMesh(
        core_axis_name="c", subcore_axis_name="s",
        num_cores=NC, num_subcores=NS)

    @functools.partial(
        pl.kernel,
        out_type=jax.ShapeDtypeStruct((NW, D, BC), jnp.float32),
        mesh=mesh,
        scratch_types=[
            pltpu.VMEM((D, BC), jnp.float32),
            pltpu.VMEM((D, BC), jnp.float32),
            pltpu.VMEM((D, BC), jnp.float32),
            pltpu.VMEM((NNZ,), jnp.int32),
            pltpu.VMEM((NNZ,), jnp.int32),
            pltpu.VMEM((NNZ,), jnp.int32),
            pltpu.VMEM((NNZ,), jnp.float32),
        ],
    )
    def sc_kernel(u1_hbm, u2_hbm, i_hbm, j_hbm, k_hbm, c_hbm, out_hbm,
                  u1_v, u2_v, o_v, i_v, j_v, k_v, c_v):
        wid = lax.axis_index("s") * NC + lax.axis_index("c")
        pltpu.sync_copy(u1_hbm.at[wid], u1_v)
        pltpu.sync_copy(u2_hbm.at[wid], u2_v)
        pltpu.sync_copy(i_hbm, i_v)
        pltpu.sync_copy(j_hbm, j_v)
        pltpu.sync_copy(k_hbm, k_v)
        pltpu.sync_copy(c_hbm, c_v)

        def zero_body(d, carry):
            for t in range(T):
                o_v[d, pl.ds(t * L, L)] = jnp.zeros((L,), jnp.float32)
            return carry

        lax.fori_loop(0, D, zero_body, 0)

        def body(n, carry):
            prev_k, acc = carry
            i = i_v[n]
            j = j_v[n]
            kk = k_v[n]
            c = c_v[n]
            is_new = kk != prev_k

            @pl.when(is_new)
            def _flush():
                for t in range(T):
                    o_v[prev_k, pl.ds(t * L, L)] = acc[t]

            cvec = jnp.broadcast_to(c, (L,))
            sel = jnp.broadcast_to(is_new, (L,))
            new_acc = []
            for t in range(T):
                a = u1_v[i, pl.ds(t * L, L)]
                b = u2_v[j, pl.ds(t * L, L)]
                p = a * b * cvec
                new_acc.append(jnp.where(sel, p, acc[t] + p))
            return kk, tuple(new_acc)

        zeros = tuple(jnp.zeros((L,), jnp.float32) for _ in range(T))
        last_k, last_acc = lax.fori_loop(0, NNZ, body, (k_v[0], zeros))
        for t in range(T):
            o_v[last_k, pl.ds(t * L, L)] = last_acc[t]

        pltpu.sync_copy(o_v, out_hbm.at[wid])

    return sc_kernel(u1w, u2w, idx_i, idx_j, idx_k, coef)


def _sym_scalar_tc(v1, v2):
    B, D = v1.shape
    blk = 256

    def body(v1_ref, v2_ref, sym_ref, sc_ref):
        p = v1_ref[...] * v2_ref[...]
        sym_ref[...] = p
        sc_ref[...] = jnp.sum(p, axis=-1, keepdims=True)

    return pl.pallas_call(
        body,
        grid=(B // blk,),
        in_specs=[
            pl.BlockSpec((blk, D), lambda b: (b, 0)),
            pl.BlockSpec((blk, D), lambda b: (b, 0)),
        ],
        out_specs=[
            pl.BlockSpec((blk, D), lambda b: (b, 0)),
            pl.BlockSpec((blk, 1), lambda b: (b, 0)),
        ],
        out_shape=[
            jax.ShapeDtypeStruct((B, D), jnp.float32),
            jax.ShapeDtypeStruct((B, 1), jnp.float32),
        ],
    )(v1, v2)


def kernel(v1, v2, I, J, K, C):
    B, D = v1.shape
    NNZ = I.shape[0]
    BC = B // NW

    u1w = jnp.transpose(v1.reshape(NW, BC, D), (0, 2, 1))
    u2w = jnp.transpose(v2.reshape(NW, BC, D), (0, 2, 1))
    outw = _bracket_sc(
        u1w, u2w,
        I.astype(jnp.int32), J.astype(jnp.int32), K.astype(jnp.int32),
        C.astype(jnp.float32), D, BC, NNZ)
    antisym = jnp.transpose(outw, (0, 2, 1)).reshape(B, D)

    sym, scalar = _sym_scalar_tc(v1, v2)
    return (antisym, sym, scalar)


# speedup vs baseline: 2.2619x; 2.2619x over previous
import functools

import jax
import jax.numpy as jnp
from jax import lax
from jax.experimental import pallas as pl
from jax.experimental.pallas import tpu as pltpu
from jax.experimental.pallas import tpu_sc as plsc

NC, NS, L = 2, 16, 16
NW = NC * NS


def _bracket_sc(u1w, u2w, idx_i, idx_j, idx_k, coef, D, BC, NNZ):
    T = BC // L
    mesh = plsc.VectorSubcoreMesh(
        core_axis_name="c", subcore_axis_name="s",
        num_cores=NC, num_subcores=NS)

    @functools.partial(
        pl.kernel,
        out_type=jax.ShapeDtypeStruct((NW, D * BC), jnp.float32),
        mesh=mesh,
        scratch_types=[
            pltpu.VMEM((D * BC,), jnp.float32),
            pltpu.VMEM((D * BC,), jnp.float32),
            pltpu.VMEM((D * BC,), jnp.float32),
            pltpu.VMEM((NNZ,), jnp.int32),
            pltpu.VMEM((NNZ,), jnp.int32),
            pltpu.VMEM((NNZ,), jnp.int32),
            pltpu.VMEM((NNZ,), jnp.float32),
        ],
    )
    def sc_kernel(u1_hbm, u2_hbm, i_hbm, j_hbm, k_hbm, c_hbm, out_hbm,
                  u1_v, u2_v, o_v, i_v, j_v, k_v, c_v):
        wid = lax.axis_index("s") * NC + lax.axis_index("c")
        pltpu.sync_copy(u1_hbm.at[wid], u1_v)
        pltpu.sync_copy(u2_hbm.at[wid], u2_v)
        pltpu.sync_copy(i_hbm, i_v)
        pltpu.sync_copy(j_hbm, j_v)
        pltpu.sync_copy(k_hbm, k_v)
        pltpu.sync_copy(c_hbm, c_v)

        def zero_body(d, carry):
            for t in range(T):
                o_v[pl.ds(d * BC + t * L, L)] = jnp.zeros((L,), jnp.float32)
            return carry

        lax.fori_loop(0, D, zero_body, 0)

        def body(g, carry):
            prev_k, acc = carry
            base = g * L
            iv = i_v[pl.ds(base, L)]
            jv = j_v[pl.ds(base, L)]
            kv = k_v[pl.ds(base, L)]
            cv = c_v[pl.ds(base, L)]
            for m in range(L):
                i = iv[m] * BC
                j = jv[m] * BC
                kk = kv[m]
                is_new = kk != prev_k

                @pl.when(is_new)
                def _flush(prev_k=prev_k, acc=acc):
                    for t in range(T):
                        o_v[pl.ds(prev_k * BC + t * L, L)] = acc[t]

                cvec = jnp.broadcast_to(cv[m], (L,))
                keep = jnp.broadcast_to(
                    jnp.where(is_new, jnp.float32(0), jnp.float32(1)), (L,))
                new_acc = []
                for t in range(T):
                    a = u1_v[pl.ds(i + t * L, L)]
                    b = u2_v[pl.ds(j + t * L, L)]
                    p = a * b * cvec
                    new_acc.append(acc[t] * keep + p)
                acc = tuple(new_acc)
                prev_k = kk
            return prev_k, acc

        zeros = tuple(jnp.zeros((L,), jnp.float32) for _ in range(T))
        k0 = k_v[pl.ds(0, L)][0]
        last_k, last_acc = lax.fori_loop(0, NNZ // L, body, (k0, zeros))
        for t in range(T):
            o_v[pl.ds(last_k * BC + t * L, L)] = last_acc[t]

        pltpu.sync_copy(o_v, out_hbm.at[wid])

    return sc_kernel(u1w, u2w, idx_i, idx_j, idx_k, coef)


def _sym_scalar_tc(v1, v2):
    B, D = v1.shape
    blk = 256

    def body(v1_ref, v2_ref, sym_ref, sc_ref):
        p = v1_ref[...] * v2_ref[...]
        sym_ref[...] = p
        sc_ref[...] = jnp.sum(p, axis=-1, keepdims=True)

    return pl.pallas_call(
        body,
        grid=(B // blk,),
        in_specs=[
            pl.BlockSpec((blk, D), lambda b: (b, 0)),
            pl.BlockSpec((blk, D), lambda b: (b, 0)),
        ],
        out_specs=[
            pl.BlockSpec((blk, D), lambda b: (b, 0)),
            pl.BlockSpec((blk, 1), lambda b: (b, 0)),
        ],
        out_shape=[
            jax.ShapeDtypeStruct((B, D), jnp.float32),
            jax.ShapeDtypeStruct((B, 1), jnp.float32),
        ],
    )(v1, v2)


def kernel(v1, v2, I, J, K, C):
    B, D = v1.shape
    NNZ = I.shape[0]
    BC = B // NW

    I = I.astype(jnp.int32)
    J = J.astype(jnp.int32)
    K = K.astype(jnp.int32)
    C = C.astype(jnp.float32)
    pad = (-NNZ) % L
    if pad:
        I = jnp.concatenate([I, jnp.zeros((pad,), jnp.int32)])
        J = jnp.concatenate([J, jnp.zeros((pad,), jnp.int32)])
        K = jnp.concatenate([K, jnp.broadcast_to(K[-1], (pad,))])
        C = jnp.concatenate([C, jnp.zeros((pad,), jnp.float32)])
        NNZ += pad

    u1w = jnp.transpose(v1.reshape(NW, BC, D), (0, 2, 1)).reshape(NW, D * BC)
    u2w = jnp.transpose(v2.reshape(NW, BC, D), (0, 2, 1)).reshape(NW, D * BC)
    outw = _bracket_sc(u1w, u2w, I, J, K, C, D, BC, NNZ)
    antisym = jnp.transpose(outw.reshape(NW, D, BC), (0, 2, 1)).reshape(B, D)

    sym, scalar = _sym_scalar_tc(v1, v2)
    return (antisym, sym, scalar)
